# baseline (speedup 1.0000x reference)
import functools

import jax
import jax.numpy as jnp
from jax import lax
from jax.experimental import pallas as pl
from jax.experimental.pallas import tpu as pltpu
from jax.experimental.pallas import tpu_sc as plsc

_N = 10000
_E = 320000
_D = 128
_G = 64
_NL = 3
_SEQ = 20

_NC = 2
_NS = 16
_DH = _D // _NC
_EW = _E // _NS
_C = 80
_NCH = _EW // _C
_RPS = 624
_RTAIL = _N - _RPS * _NS


def _bn_cols(h, g, b):
    m = jnp.mean(h, axis=0, keepdims=True)
    v = jnp.mean((h - m) ** 2, axis=0, keepdims=True)
    return g * (h - m) / jnp.sqrt(v + 1e-5) + b


def _stage_a_body(x_ref, winT_ref, bin_ref, gin_ref, betain_ref, wcxT_ref,
                  bc_ref, h0_ref, hw_ref, self_ref):
    h = jnp.maximum(
        jnp.dot(x_ref[...], winT_ref[...], preferred_element_type=jnp.float32)
        + bin_ref[...], 0.0)
    h0 = _bn_cols(h, gin_ref[...], betain_ref[...])
    hw = jnp.dot(h0, wcxT_ref[...], preferred_element_type=jnp.float32)
    h0_ref[...] = h0
    hwb = hw + bc_ref[...]
    hw_ref[...] = hwb
    sm = jnp.maximum(hwb, 0.0)
    self_ref[0] = sm[:, :_DH]
    self_ref[1] = sm[:, _DH:]


def _stage_b_body(parts_ref, hsum_ref, g_ref, b_ref, wcxT_ref,
                  bc_ref, hsum_out_ref, hw_ref, selfn_ref):
    agg = jnp.concatenate([parts_ref[0], parts_ref[1]], axis=1)
    h = _bn_cols(agg, g_ref[...], b_ref[...])
    hsum_out_ref[...] = hsum_ref[...] + h
    hwb = jnp.dot(h, wcxT_ref[...],
                  preferred_element_type=jnp.float32) + bc_ref[...]
    hw_ref[...] = hwb
    sm = jnp.maximum(hwb, 0.0)
    selfn_ref[0] = sm[:, :_DH]
    selfn_ref[1] = sm[:, _DH:]


def _stage_b_last_body(parts_ref, hsum_ref, g_ref, b_ref, hsum_out_ref):
    agg = jnp.concatenate([parts_ref[0], parts_ref[1]], axis=1)
    h = _bn_cols(agg, g_ref[...], b_ref[...])
    hsum_out_ref[...] = hsum_ref[...] + h


def _ew_body(ea_ref, w_ref, out_ref):
    out_ref[0] = jnp.dot(ea_ref[...], w_ref[0],
                         preferred_element_type=jnp.float32)


def _stage_c_body(hsum_ref, bid_ref, wspT_ref, bsp_ref, gsp_ref, betasp_ref,
                  wpT_ref, bp_ref, hs_ref, pool_ref):
    hsum = hsum_ref[...]
    bid = bid_ref[...]
    gids = lax.broadcasted_iota(jnp.int32, (1, _G), 1)
    onehot = (bid == gids).astype(jnp.float32)
    ones = jnp.ones((_N, 1), jnp.float32)
    cnt = lax.dot_general(onehot, ones, (((0,), (0,)), ((), ())),
                          preferred_element_type=jnp.float32)
    xadd = lax.dot_general(onehot, hsum, (((0,), (0,)), ((), ())),
                           preferred_element_type=jnp.float32)
    xmean = xadd / jnp.maximum(cnt, 1.0)
    neg = jnp.full_like(hsum, -jnp.inf)
    rows = []
    for g in range(_G):
        mg = bid == g
        rows.append(jnp.max(jnp.where(mg, hsum, neg), axis=0, keepdims=True))
    xmax = jnp.concatenate(rows, axis=0)
    xc = jnp.concatenate([xmean, xadd, xmax], axis=1)
    hsv = jnp.maximum(
        jnp.dot(xc, wspT_ref[...], preferred_element_type=jnp.float32)
        + bsp_ref[...], 0.0)
    hs = _bn_cols(hsv, gsp_ref[...], betasp_ref[...])
    hs_ref[...] = hs
    pool_ref[...] = jnp.tanh(
        jnp.dot(hs, wpT_ref[...], preferred_element_type=jnp.float32)
        + bp_ref[...])


def _stage_d_body(bid_ref, pool_ref, wd1T_ref, bd1_ref, gd1_ref, betad1_ref,
                  wd2T_ref, bd2_ref, rec_ref):
    bid = bid_ref[...]
    gids = lax.broadcasted_iota(jnp.int32, (1, _G), 1)
    onehot = (bid == gids).astype(jnp.float32)
    z = jnp.dot(onehot, pool_ref[...], preferred_element_type=jnp.float32)
    hd = _bn_cols(
        jnp.maximum(
            jnp.dot(z, wd1T_ref[...], preferred_element_type=jnp.float32)
            + bd1_ref[...], 0.0),
        gd1_ref[...], betad1_ref[...])
    rec_ref[...] = (jnp.dot(hd, wd2T_ref[...],
                            preferred_element_type=jnp.float32)
                    + bd2_ref[...])


_NB = 4
_KQ = (_NCH - 2) // _NB
_NTAIL = _NCH - _KQ * _NB


def _make_edge_kernel():
    mesh = plsc.VectorSubcoreMesh(core_axis_name="c", subcore_axis_name="s")

    @functools.partial(
        pl.kernel, mesh=mesh,
        compiler_params=pltpu.CompilerParams(use_tc_tiling_on_sc=False),
        out_type=jax.ShapeDtypeStruct((_NC, _N, _DH), jnp.float32),
        scratch_types=[
            pltpu.VMEM((_NB, _C), jnp.int32),
            pltpu.VMEM((_NB, _C), jnp.int32),
            pltpu.VMEM((_NB, _C, _DH), jnp.float32),
            pltpu.VMEM((_NB, _C, _DH), jnp.float32),
            pltpu.VMEM_SHARED((_N, _DH), jnp.float32),
        ] + [pltpu.SemaphoreType.DMA] * (5 * _NB),
    )
    def edge_kernel(hwi_hbm, selfmsg_hbm, ew_hbm, src_hbm, dst_hbm,
                    out_hbm, srcv, dstv, ewv, rows, aggs, *sems):
        isems = sems[0 * _NB:1 * _NB]
        esems = sems[1 * _NB:2 * _NB]
        dsems = sems[2 * _NB:3 * _NB]
        gsems = sems[3 * _NB:4 * _NB]
        ssems = sems[4 * _NB:5 * _NB]
        cid = lax.axis_index("c")
        sid = lax.axis_index("s")
        roff = pl.multiple_of(sid * _RPS, 8)
        pltpu.sync_copy(selfmsg_hbm.at[cid, pl.ds(roff, _RPS), :],
                        aggs.at[pl.ds(roff, _RPS), :])

        @pl.when(sid == _NS - 1)
        def _init_tail():
            pltpu.sync_copy(selfmsg_hbm.at[cid, pl.ds(_RPS * _NS, _RTAIL), :],
                            aggs.at[pl.ds(_RPS * _NS, _RTAIL), :])

        plsc.subcore_barrier()

        base_w = sid * _EW

        def off(c):
            return pl.multiple_of(base_w + c * _C, 8)

        def issue_srcea(c, b):
            o = off(c)
            pltpu.async_copy(src_hbm.at[pl.ds(o, _C)], srcv.at[b], isems[b])
            pltpu.async_copy(ew_hbm.at[cid, pl.ds(o, _C), :], ewv.at[b],
                             esems[b])

        def wait_src(b):
            pltpu.make_async_copy(src_hbm.at[pl.ds(0, _C)], srcv.at[b],
                                  isems[b]).wait()

        def xform_src(b):
            for t in range(_C // 16):
                sl = pl.ds(16 * t, 16)
                srcv[b, sl] = srcv[b, sl] * 2 + cid

        def wait_ea(b):
            pltpu.make_async_copy(ew_hbm.at[cid, pl.ds(0, _C), :], ewv.at[b],
                                  esems[b]).wait()

        def issue_dst(c, b):
            pltpu.async_copy(dst_hbm.at[pl.ds(off(c), _C)], dstv.at[b],
                             dsems[b])

        def wait_dst(b):
            pltpu.make_async_copy(dst_hbm.at[pl.ds(0, _C)], dstv.at[b],
                                  dsems[b]).wait()

        def issue_gather(b):
            pltpu.async_copy(hwi_hbm.at[srcv.at[b]], rows.at[b], gsems[b])

        def wait_gather(b):
            pltpu.make_async_copy(hwi_hbm.at[srcv.at[b]], rows.at[b],
                                  gsems[b]).wait()

        def issue_scatter(b):
            pltpu.async_copy(rows.at[b], aggs.at[dstv.at[b]], ssems[b],
                             add=True)

        def wait_scatter(b):
            pltpu.make_async_copy(rows.at[b], aggs.at[dstv.at[b]],
                                  ssems[b]).wait()

        def compute(b):
            @plsc.parallel_loop(0, _C, step=1, unroll=4)
            def edge_body(e):
                for j in range(_DH // 16):
                    sl = pl.ds(16 * j, 16)
                    rows[b, e, sl] = jnp.maximum(
                        rows[b, e, sl] + ewv[b, e, sl], 0.0)

        for b in range(_NB):
            issue_srcea(b, b)
            issue_dst(b, b)
        for b in range(_NB):
            wait_src(b)
            xform_src(b)
            issue_gather(b)

        def quad_body(k, carry):
            more = k < _KQ - 1
            for b in range(_NB):
                wait_gather(b)
                wait_ea(b)
                compute(b)

                @pl.when(more)
                def _prefetch():
                    issue_srcea(_NB * k + _NB + b, b)

                wait_dst(b)
                issue_scatter(b)

            @pl.when(more)
            def _next_gathers():
                for b in range(_NB):
                    wait_scatter(b)
                    issue_dst(_NB * k + _NB + b, b)
                    wait_src(b)
                    xform_src(b)
                    issue_gather(b)

            return carry

        lax.fori_loop(0, _KQ, quad_body, 0)
        for b in range(_NB):
            wait_scatter(b)

        for t in range(_NTAIL):
            c = _KQ * _NB + t
            issue_srcea(c, 0)
            issue_dst(c, 0)
            wait_src(0)
            xform_src(0)
            issue_gather(0)
            wait_ea(0)
            wait_gather(0)
            compute(0)
            wait_dst(0)
            issue_scatter(0)
            wait_scatter(0)

        plsc.subcore_barrier()
        pltpu.sync_copy(aggs.at[pl.ds(roff, _RPS), :],
                        out_hbm.at[cid, pl.ds(roff, _RPS), :])

        @pl.when(sid == _NS - 1)
        def _out_tail():
            pltpu.sync_copy(aggs.at[pl.ds(_RPS * _NS, _RTAIL), :],
                            out_hbm.at[cid, pl.ds(_RPS * _NS, _RTAIL), :])

    return edge_kernel


_NDOUT = jax.ShapeDtypeStruct((_N, _D), jnp.float32)
_GDOUT = jax.ShapeDtypeStruct((_G, _D), jnp.float32)


def kernel(x, edge_index, edge_attr, batch_idx, Win, bin_, gin, betain,
           Wc, bc, gc, betac, Wsp, bsp, gsp, betasp, Wp, bp,
           Wd1, bd1, gd1, betad1, Wd2, bd2):
    row = lambda v: v.reshape(1, -1)
    src = edge_index[0]
    dst = edge_index[1]
    bid2 = batch_idx.reshape(_N, 1)
    WcxT = [Wc[l, :, :_D].T for l in range(_NL)]
    WceT = [Wc[l, :, _D:].T for l in range(_NL)]
    _EB = 4000
    WceS = [jnp.stack([WceT[l][:, :_DH], WceT[l][:, _DH:]])
            for l in range(_NL)]
    ews = [pl.pallas_call(
        _ew_body,
        grid=(_NC, _E // _EB),
        in_specs=[pl.BlockSpec((_EB, 6), lambda h, i: (i, 0)),
                  pl.BlockSpec((1, 6, _DH), lambda h, i: (h, 0, 0))],
        out_specs=pl.BlockSpec((1, _EB, _DH), lambda h, i: (h, i, 0)),
        out_shape=jax.ShapeDtypeStruct((_NC, _E, _DH), jnp.float32),
    )(edge_attr, WceS[l]) for l in range(_NL)]

    _SELF2 = jax.ShapeDtypeStruct((_NC, _N, _DH), jnp.float32)
    h0, hw, selfmsg = pl.pallas_call(
        _stage_a_body,
        out_shape=[_NDOUT, _NDOUT, _SELF2],
    )(x, Win.T, row(bin_), row(gin), row(betain), WcxT[0], row(bc[0]))
    hsum = h0

    edge_call = _make_edge_kernel()
    for l in range(_NL):
        hwi = hw.reshape(2 * _N, _DH)
        parts = edge_call(hwi, selfmsg, ews[l], src, dst)
        if l < _NL - 1:
            hsum, hw, selfmsg = pl.pallas_call(
                _stage_b_body,
                out_shape=[_NDOUT, _NDOUT, _SELF2],
            )(parts, hsum, row(gc[l]), row(betac[l]),
              WcxT[l + 1], row(bc[l + 1]))
        else:
            hsum = pl.pallas_call(
                _stage_b_last_body,
                out_shape=_NDOUT,
            )(parts, hsum, row(gc[l]), row(betac[l]))

    hs, pooler = pl.pallas_call(
        _stage_c_body,
        out_shape=[_GDOUT, _GDOUT],
    )(hsum, bid2, Wsp.T, row(bsp), row(gsp), row(betasp), Wp.T, row(bp))

    reconstructed = pl.pallas_call(
        _stage_d_body,
        out_shape=_NDOUT,
    )(bid2, pooler, Wd1.T, row(bd1), row(gd1), row(betad1), Wd2.T, row(bd2))

    last_hidden_state = jnp.broadcast_to(hs[:, None, :], (_G, _SEQ, _D))
    return (last_hidden_state, pooler, reconstructed)


# speedup vs baseline: 1.9019x; 1.9019x over previous
import functools

import jax
import jax.numpy as jnp
from jax import lax
from jax.experimental import pallas as pl
from jax.experimental.pallas import tpu as pltpu
from jax.experimental.pallas import tpu_sc as plsc

_N = 10000
_E = 320000
_D = 128
_G = 64
_NL = 3
_SEQ = 20

_NC = 2
_NS = 16
_NW = _NC * _NS
_EW = _E // _NW
_C = 80
_NCH = _EW // _C
_RPS = 624
_RTAIL = _N - _RPS * _NS
_NB = 2
_KP = (_NCH - 1) // _NB
_NTAIL = _NCH - _KP * _NB


def _bn_cols(h, g, b):
    m = jnp.mean(h, axis=0, keepdims=True)
    v = jnp.mean((h - m) ** 2, axis=0, keepdims=True)
    return g * (h - m) / jnp.sqrt(v + 1e-5) + b


def _stage_a_body(x_ref, winT_ref, bin_ref, gin_ref, betain_ref, wcxT_ref,
                  bc_ref, h0_ref, hw_ref, self_ref):
    h = jnp.maximum(
        jnp.dot(x_ref[...], winT_ref[...], preferred_element_type=jnp.float32)
        + bin_ref[...], 0.0)
    h0 = _bn_cols(h, gin_ref[...], betain_ref[...])
    hw = jnp.dot(h0, wcxT_ref[...], preferred_element_type=jnp.float32)
    h0_ref[...] = h0
    hwb = hw + bc_ref[...]
    hw_ref[...] = hwb
    self_ref[...] = jnp.maximum(hwb, 0.0)


def _stage_b_body(parts_ref, self_ref, hsum_ref, g_ref, b_ref, wcxT_ref,
                  bc_ref, hsum_out_ref, hw_ref, selfn_ref):
    agg = parts_ref[0] + parts_ref[1] - self_ref[...]
    h = _bn_cols(agg, g_ref[...], b_ref[...])
    hsum_out_ref[...] = hsum_ref[...] + h
    hwb = jnp.dot(h, wcxT_ref[...],
                  preferred_element_type=jnp.float32) + bc_ref[...]
    hw_ref[...] = hwb
    selfn_ref[...] = jnp.maximum(hwb, 0.0)


def _stage_b_last_body(parts_ref, self_ref, hsum_ref, g_ref, b_ref,
                       hsum_out_ref):
    agg = parts_ref[0] + parts_ref[1] - self_ref[...]
    h = _bn_cols(agg, g_ref[...], b_ref[...])
    hsum_out_ref[...] = hsum_ref[...] + h


def _ew_body(ea_ref, w_ref, out_ref):
    out_ref[...] = jnp.dot(ea_ref[...], w_ref[...],
                           preferred_element_type=jnp.float32)


def _stage_c_body(hsum_ref, bid_ref, wspT_ref, bsp_ref, gsp_ref, betasp_ref,
                  wpT_ref, bp_ref, hs_ref, pool_ref):
    hsum = hsum_ref[...]
    bid = bid_ref[...]
    gids = lax.broadcasted_iota(jnp.int32, (1, _G), 1)
    onehot = (bid == gids).astype(jnp.float32)
    ones = jnp.ones((_N, 1), jnp.float32)
    cnt = lax.dot_general(onehot, ones, (((0,), (0,)), ((), ())),
                          preferred_element_type=jnp.float32)
    xadd = lax.dot_general(onehot, hsum, (((0,), (0,)), ((), ())),
                           preferred_element_type=jnp.float32)
    xmean = xadd / jnp.maximum(cnt, 1.0)
    neg = jnp.full_like(hsum, -jnp.inf)
    rows = []
    for g in range(_G):
        mg = bid == g
        rows.append(jnp.max(jnp.where(mg, hsum, neg), axis=0, keepdims=True))
    xmax = jnp.concatenate(rows, axis=0)
    xc = jnp.concatenate([xmean, xadd, xmax], axis=1)
    hsv = jnp.maximum(
        jnp.dot(xc, wspT_ref[...], preferred_element_type=jnp.float32)
        + bsp_ref[...], 0.0)
    hs = _bn_cols(hsv, gsp_ref[...], betasp_ref[...])
    hs_ref[...] = hs
    pool_ref[...] = jnp.tanh(
        jnp.dot(hs, wpT_ref[...], preferred_element_type=jnp.float32)
        + bp_ref[...])


def _stage_d_body(bid_ref, pool_ref, wd1T_ref, bd1_ref, gd1_ref, betad1_ref,
                  wd2T_ref, bd2_ref, rec_ref):
    bid = bid_ref[...]
    gids = lax.broadcasted_iota(jnp.int32, (1, _G), 1)
    onehot = (bid == gids).astype(jnp.float32)
    z = jnp.dot(onehot, pool_ref[...], preferred_element_type=jnp.float32)
    hd = _bn_cols(
        jnp.maximum(
            jnp.dot(z, wd1T_ref[...], preferred_element_type=jnp.float32)
            + bd1_ref[...], 0.0),
        gd1_ref[...], betad1_ref[...])
    rec_ref[...] = (jnp.dot(hd, wd2T_ref[...],
                            preferred_element_type=jnp.float32)
                    + bd2_ref[...])


def _make_edge_kernel():
    mesh = plsc.VectorSubcoreMesh(core_axis_name="c", subcore_axis_name="s")

    @functools.partial(
        pl.kernel, mesh=mesh,
        out_type=jax.ShapeDtypeStruct((_NC, _N, _D), jnp.float32),
        scratch_types=[
            pltpu.VMEM((_NB, _C), jnp.int32),
            pltpu.VMEM((_NB, _C), jnp.int32),
            pltpu.VMEM((_NB, _C, _D), jnp.float32),
            pltpu.VMEM((_NB, _C, _D), jnp.float32),
            pltpu.VMEM_SHARED((_N, _D), jnp.float32),
        ] + [pltpu.SemaphoreType.DMA] * (5 * _NB),
    )
    def edge_kernel(hw_hbm, selfmsg_hbm, ew_hbm, src_hbm, dst_hbm,
                    out_hbm, srcv, dstv, ewv, rows, aggs, *sems):
        isems = sems[0 * _NB:1 * _NB]
        esems = sems[1 * _NB:2 * _NB]
        dsems = sems[2 * _NB:3 * _NB]
        gsems = sems[3 * _NB:4 * _NB]
        ssems = sems[4 * _NB:5 * _NB]
        cid = lax.axis_index("c")
        sid = lax.axis_index("s")
        w = cid * _NS + sid
        roff = pl.multiple_of(sid * _RPS, 8)
        pltpu.sync_copy(selfmsg_hbm.at[pl.ds(roff, _RPS), :],
                        aggs.at[pl.ds(roff, _RPS), :])

        @pl.when(sid == _NS - 1)
        def _init_tail():
            pltpu.sync_copy(selfmsg_hbm.at[pl.ds(_RPS * _NS, _RTAIL), :],
                            aggs.at[pl.ds(_RPS * _NS, _RTAIL), :])

        plsc.subcore_barrier()

        base_w = w * _EW

        def off(c):
            return pl.multiple_of(base_w + c * _C, 8)

        def issue_srcea(c, b):
            o = off(c)
            pltpu.async_copy(src_hbm.at[pl.ds(o, _C)], srcv.at[b], isems[b])
            pltpu.async_copy(ew_hbm.at[pl.ds(o, _C), :], ewv.at[b], esems[b])

        def wait_src(b):
            pltpu.make_async_copy(src_hbm.at[pl.ds(0, _C)], srcv.at[b],
                                  isems[b]).wait()

        def wait_ea(b):
            pltpu.make_async_copy(ew_hbm.at[pl.ds(0, _C), :], ewv.at[b],
                                  esems[b]).wait()

        def issue_dst(c, b):
            pltpu.async_copy(dst_hbm.at[pl.ds(off(c), _C)], dstv.at[b],
                             dsems[b])

        def wait_dst(b):
            pltpu.make_async_copy(dst_hbm.at[pl.ds(0, _C)], dstv.at[b],
                                  dsems[b]).wait()

        def issue_gather(b):
            pltpu.async_copy(hw_hbm.at[srcv.at[b]], rows.at[b], gsems[b])

        def wait_gather(b):
            pltpu.make_async_copy(hw_hbm.at[srcv.at[b]], rows.at[b],
                                  gsems[b]).wait()

        def issue_scatter(b):
            pltpu.async_copy(rows.at[b], aggs.at[dstv.at[b]], ssems[b],
                             add=True)

        def wait_scatter(b):
            pltpu.make_async_copy(rows.at[b], aggs.at[dstv.at[b]],
                                  ssems[b]).wait()

        def compute(b):
            @plsc.parallel_loop(0, _C, step=1, unroll=4)
            def edge_body(e):
                for j in range(_D // 16):
                    sl = pl.ds(16 * j, 16)
                    rows[b, e, sl] = jnp.maximum(
                        rows[b, e, sl] + ewv[b, e, sl], 0.0)

        for b in range(_NB):
            issue_srcea(b, b)
            issue_dst(b, b)
        for b in range(_NB):
            wait_src(b)
            issue_gather(b)

        def pair_body(k, carry):
            more = k < _KP - 1
            for b in range(_NB):
                wait_gather(b)
                wait_ea(b)
                compute(b)

                @pl.when(more)
                def _prefetch():
                    issue_srcea(_NB * k + _NB + b, b)

                wait_dst(b)
                issue_scatter(b)

            @pl.when(more)
            def _next_gathers():
                for b in range(_NB):
                    wait_scatter(b)
                    issue_dst(_NB * k + _NB + b, b)
                    wait_src(b)
                    issue_gather(b)

            return carry

        lax.fori_loop(0, _KP, pair_body, 0)
        for b in range(_NB):
            wait_scatter(b)

        for t in range(_NTAIL):
            c = _KP * _NB + t
            issue_srcea(c, 0)
            issue_dst(c, 0)
            wait_src(0)
            issue_gather(0)
            wait_ea(0)
            wait_gather(0)
            compute(0)
            wait_dst(0)
            issue_scatter(0)
            wait_scatter(0)

        plsc.subcore_barrier()
        pltpu.sync_copy(aggs.at[pl.ds(roff, _RPS), :],
                        out_hbm.at[cid, pl.ds(roff, _RPS), :])

        @pl.when(sid == _NS - 1)
        def _out_tail():
            pltpu.sync_copy(aggs.at[pl.ds(_RPS * _NS, _RTAIL), :],
                            out_hbm.at[cid, pl.ds(_RPS * _NS, _RTAIL), :])

    return edge_kernel


_NDOUT = jax.ShapeDtypeStruct((_N, _D), jnp.float32)
_GDOUT = jax.ShapeDtypeStruct((_G, _D), jnp.float32)


def kernel(x, edge_index, edge_attr, batch_idx, Win, bin_, gin, betain,
           Wc, bc, gc, betac, Wsp, bsp, gsp, betasp, Wp, bp,
           Wd1, bd1, gd1, betad1, Wd2, bd2):
    row = lambda v: v.reshape(1, -1)
    src = edge_index[0]
    dst = edge_index[1]
    bid2 = batch_idx.reshape(_N, 1)
    WcxT = [Wc[l, :, :_D].T for l in range(_NL)]
    WceT = [Wc[l, :, _D:].T for l in range(_NL)]

    _EB = 4000
    ews = [pl.pallas_call(
        _ew_body,
        grid=(_E // _EB,),
        in_specs=[pl.BlockSpec((_EB, 6), lambda i: (i, 0)),
                  pl.BlockSpec((6, _D), lambda i: (0, 0))],
        out_specs=pl.BlockSpec((_EB, _D), lambda i: (i, 0)),
        out_shape=jax.ShapeDtypeStruct((_E, _D), jnp.float32),
    )(edge_attr, WceT[l]) for l in range(_NL)]

    h0, hw, selfmsg = pl.pallas_call(
        _stage_a_body,
        out_shape=[_NDOUT, _NDOUT, _NDOUT],
    )(x, Win.T, row(bin_), row(gin), row(betain), WcxT[0], row(bc[0]))
    hsum = h0

    edge_call = _make_edge_kernel()
    for l in range(_NL):
        parts = edge_call(hw, selfmsg, ews[l], src, dst)
        if l < _NL - 1:
            hsum, hw, selfmsg = pl.pallas_call(
                _stage_b_body,
                out_shape=[_NDOUT, _NDOUT, _NDOUT],
            )(parts, selfmsg, hsum, row(gc[l]), row(betac[l]),
              WcxT[l + 1], row(bc[l + 1]))
        else:
            hsum = pl.pallas_call(
                _stage_b_last_body,
                out_shape=_NDOUT,
            )(parts, selfmsg, hsum, row(gc[l]), row(betac[l]))

    hs, pooler = pl.pallas_call(
        _stage_c_body,
        out_shape=[_GDOUT, _GDOUT],
    )(hsum, bid2, Wsp.T, row(bsp), row(gsp), row(betasp), Wp.T, row(bp))

    reconstructed = pl.pallas_call(
        _stage_d_body,
        out_shape=_NDOUT,
    )(bid2, pooler, Wd1.T, row(bd1), row(gd1), row(betad1), Wd2.T, row(bd2))

    last_hidden_state = jnp.broadcast_to(hs[:, None, :], (_G, _SEQ, _D))
    return (last_hidden_state, pooler, reconstructed)
